# baseline (speedup 1.0000x reference)
import jax
import jax.numpy as jnp
from jax.experimental import pallas as pl
from jax.experimental.pallas import tpu as pltpu

B = 128
DEPTH = 9
NODES = 2 ** (DEPTH + 1) - 1
LEAVES = 2 ** DEPTH
N_FEAT = 4
N_HID = 64
JF = 4
W = JF * N_HID
G = 8
JPP = JF * G
NPROG = B // JPP

_PAD = [max(2 ** d, 8) for d in range(DEPTH + 1)]
_OFF = [sum(_PAD[:d]) for d in range(DEPTH + 1)]
NPADDED = sum(_PAD)
_INNER_PAD = _OFF[DEPTH]

_bf = jnp.bfloat16
_f32 = jnp.float32


def _body(c_ref, wu_ref, wlr_ref, whu_ref, bh_ref, o_ref, *scr):
    c = jnp.concatenate(
        [c_ref[:, 0, 0, :], jnp.full((NPADDED, 128), 1.0, _bf)],
        axis=1)
    u = jnp.tanh(jnp.dot(c, wu_ref[...],
                         preferred_element_type=_f32).astype(_bf))
    vs = []
    for g in range(G):
        ug = u[:_INNER_PAD, W * g:W * (g + 1)]
        vs.append(jnp.dot(ug, whu_ref[...], preferred_element_type=_f32)
                  + bh_ref[...])
    for g in range(G):
        leaves = u[_INNER_PAD:, W * g:W * (g + 1)].astype(_f32)
        scr[g][0, :, :] = leaves[:, :128]
        scr[g][1, :, :] = leaves[:, 128:]
    for d in range(DEPTH - 1, -1, -1):
        n = 2 ** d
        hh = jnp.concatenate(
            [jnp.concatenate([scr[g][0, pl.ds(0, n, 2), :] for g in range(G)],
                             axis=0),
             jnp.concatenate([scr[g][1, pl.ds(0, n, 2), :] for g in range(G)],
                             axis=0),
             jnp.concatenate([scr[g][0, pl.ds(1, n, 2), :] for g in range(G)],
                             axis=0),
             jnp.concatenate([scr[g][1, pl.ds(1, n, 2), :] for g in range(G)],
                             axis=0)], axis=1).astype(_bf)
        v_cat = jnp.concatenate(
            [vs[g][_OFF[d]:_OFF[d] + n] for g in range(G)], axis=0)
        new = jnp.tanh(
            jnp.dot(hh, wlr_ref[...], preferred_element_type=_f32)
            + v_cat)
        if d > 0:
            for g in range(G):
                scr[g][0, pl.ds(0, n), :] = new[n * g:n * (g + 1), :128]
                scr[g][1, pl.ds(0, n), :] = new[n * g:n * (g + 1), 128:]
        else:
            o_ref[...] = new.reshape(G, 1, W)


def _bdiag(x, k):
    a, b = x.shape[-2:]
    eye = jnp.eye(k, dtype=x.dtype)
    t = (eye[..., :, None, :, None] * x[..., None, :, None, :])
    return t.reshape(*x.shape[:-2], k * a, k * b)


def kernel(content, Wu, bu, Wh, bh):
    c_w = (content.reshape(B, NODES, N_FEAT).transpose(1, 0, 2)
           .reshape(NODES, B * N_FEAT).astype(_bf))
    pieces = []
    for d in range(DEPTH + 1):
        n = 2 ** d
        pieces.append(c_w[n - 1:2 * n - 1])
        if _PAD[d] > n:
            pieces.append(jnp.zeros((_PAD[d] - n, B * N_FEAT), _bf))
    c_pad = jnp.concatenate(pieces, axis=0).reshape(
        NPADDED, NPROG, 1, JPP * N_FEAT)
    Wu_bd = _bdiag(Wu.T, JPP)
    Wu_all = jnp.concatenate([
        Wu_bd,
        jnp.tile(bu, JPP).reshape(1, JPP * N_HID),
        jnp.zeros((127, JPP * N_HID), _f32)], axis=0).astype(_bf)
    BD3 = _bdiag(Wh.T.reshape(3, N_HID, N_HID), JF)
    Wlr = jnp.concatenate([BD3[0], BD3[1]], axis=0).astype(_bf)
    Whu_bd = BD3[2].astype(_bf)
    bh_t = jnp.tile(bh, JF).reshape(1, W)

    out = pl.pallas_call(
        _body,
        grid=(NPROG,),
        in_specs=[
            pl.BlockSpec((NPADDED, 1, 1, JPP * N_FEAT), lambda i: (0, i, 0, 0)),
            pl.BlockSpec((W, JPP * N_HID), lambda i: (0, 0)),
            pl.BlockSpec((2 * W, W), lambda i: (0, 0)),
            pl.BlockSpec((W, W), lambda i: (0, 0)),
            pl.BlockSpec((1, W), lambda i: (0, 0)),
        ],
        out_specs=pl.BlockSpec((G, 1, W), lambda i: (i, 0, 0)),
        out_shape=jax.ShapeDtypeStruct((NPROG * G, 1, W), jnp.float32),
        scratch_shapes=[pltpu.VMEM((2, LEAVES, 128), jnp.float32)
                        for _ in range(G)],
    )(c_pad, Wu_all, Wlr, Whu_bd, bh_t)
    return out.reshape(B, N_HID)


# speedup vs baseline: 1.1270x; 1.1270x over previous
import jax
import jax.numpy as jnp
from jax.experimental import pallas as pl
from jax.experimental.pallas import tpu as pltpu

B = 128
DEPTH = 9
NODES = 2 ** (DEPTH + 1) - 1
LEAVES = 2 ** DEPTH
N_FEAT = 4
N_HID = 64
JF = 4
W = JF * N_HID
G = 8
JPP = JF * G
NPROG = B // JPP

INNER = NODES - LEAVES

_bf = jnp.bfloat16
_f32 = jnp.float32


def _body(c_ref, wu_ref, wlr_ref, whu_ref, bh_ref, o_ref, *scr):
    ones = jnp.full((INNER, 128), 1.0, _bf)
    c_in = jnp.concatenate([c_ref[:INNER, 0, 0, :], ones], axis=1)
    c_lf = jnp.concatenate([c_ref[INNER:, 0, 0, :],
                            jnp.full((LEAVES, 128), 1.0, _bf)], axis=1)
    u_in = jnp.tanh(jnp.dot(c_in, wu_ref[...],
                            preferred_element_type=_f32).astype(_bf))
    u_lf = jnp.tanh(jnp.dot(c_lf, wu_ref[...],
                            preferred_element_type=_f32).astype(_bf))
    vs = []
    for g in range(G):
        ug = u_in[:, W * g:W * (g + 1)]
        vs.append(jnp.dot(ug, whu_ref[...], preferred_element_type=_f32)
                  + bh_ref[...])
    for g in range(G):
        leaves = u_lf[:, W * g:W * (g + 1)].astype(_f32)
        scr[g][0, :, :] = leaves[:, :128]
        scr[g][1, :, :] = leaves[:, 128:]
    for d in range(DEPTH - 1, -1, -1):
        n = 2 ** d
        hh = jnp.concatenate(
            [jnp.concatenate([scr[g][0, pl.ds(0, n, 2), :] for g in range(G)],
                             axis=0),
             jnp.concatenate([scr[g][1, pl.ds(0, n, 2), :] for g in range(G)],
                             axis=0),
             jnp.concatenate([scr[g][0, pl.ds(1, n, 2), :] for g in range(G)],
                             axis=0),
             jnp.concatenate([scr[g][1, pl.ds(1, n, 2), :] for g in range(G)],
                             axis=0)], axis=1).astype(_bf)
        v_cat = jnp.concatenate(
            [vs[g][n - 1:2 * n - 1] for g in range(G)], axis=0)
        new = jnp.tanh(
            jnp.dot(hh, wlr_ref[...], preferred_element_type=_f32)
            + v_cat)
        if d > 0:
            for g in range(G):
                scr[g][0, pl.ds(0, n), :] = new[n * g:n * (g + 1), :128]
                scr[g][1, pl.ds(0, n), :] = new[n * g:n * (g + 1), 128:]
        else:
            o_ref[...] = new.reshape(G, 1, W)


def _bdiag(x, k):
    a, b = x.shape[-2:]
    eye = jnp.eye(k, dtype=x.dtype)
    t = (eye[..., :, None, :, None] * x[..., None, :, None, :])
    return t.reshape(*x.shape[:-2], k * a, k * b)


def kernel(content, Wu, bu, Wh, bh):
    c_pad = (content.reshape(B, NODES, N_FEAT).transpose(1, 0, 2)
             .reshape(NODES, NPROG, 1, JPP * N_FEAT).astype(_bf))
    Wu_bd = _bdiag(Wu.T, JPP)
    Wu_all = jnp.concatenate([
        Wu_bd,
        jnp.tile(bu, JPP).reshape(1, JPP * N_HID),
        jnp.zeros((127, JPP * N_HID), _f32)], axis=0).astype(_bf)
    BD3 = _bdiag(Wh.T.reshape(3, N_HID, N_HID), JF)
    Wlr = jnp.concatenate([BD3[0], BD3[1]], axis=0).astype(_bf)
    Whu_bd = BD3[2].astype(_bf)
    bh_t = jnp.tile(bh, JF).reshape(1, W)

    out = pl.pallas_call(
        _body,
        grid=(NPROG,),
        in_specs=[
            pl.BlockSpec((NODES, 1, 1, JPP * N_FEAT), lambda i: (0, i, 0, 0)),
            pl.BlockSpec((W, JPP * N_HID), lambda i: (0, 0)),
            pl.BlockSpec((2 * W, W), lambda i: (0, 0)),
            pl.BlockSpec((W, W), lambda i: (0, 0)),
            pl.BlockSpec((1, W), lambda i: (0, 0)),
        ],
        out_specs=pl.BlockSpec((G, 1, W), lambda i: (i, 0, 0)),
        out_shape=jax.ShapeDtypeStruct((NPROG * G, 1, W), jnp.float32),
        scratch_shapes=[pltpu.VMEM((2, LEAVES, 128), jnp.float32)
                        for _ in range(G)],
    )(c_pad, Wu_all, Wlr, Whu_bd, bh_t)
    return out.reshape(B, N_HID)


# speedup vs baseline: 1.1743x; 1.0420x over previous
import numpy as np
import jax
import jax.numpy as jnp
from jax.experimental import pallas as pl
from jax.experimental.pallas import tpu as pltpu

B = 128
DEPTH = 9
NODES = 2 ** (DEPTH + 1) - 1
LEAVES = 2 ** DEPTH
INNER = NODES - LEAVES
N_FEAT = 4
N_HID = 64
JF = 4
W = JF * N_HID
G = 8
JPP = JF * G
NPROG = B // JPP

_bf = jnp.bfloat16
_f32 = jnp.float32


def _body(c_ref, wu_ref, bd3_ref, bu_ref, bh_ref, o_ref, *scr):
    c = c_ref[:, 0, 0, :]
    vs = []
    for g in range(G):
        u = jnp.tanh((jnp.dot(c, wu_ref[g], preferred_element_type=_f32)
                      + bu_ref[...]).astype(_bf))
        v = (jnp.dot(u[:INNER], bd3_ref[2], preferred_element_type=_f32)
             + bh_ref[...])
        leaves = u[INNER:].astype(_f32)
        scr[g][0, :, :] = leaves[:, :128]
        scr[g][1, :, :] = leaves[:, 128:]
        vs.append(v)
    new = [None] * G
    for d in range(DEPTH - 1, -1, -1):
        n = 2 ** d
        for g in range(G):
            h_l = jnp.concatenate(
                [scr[g][0, pl.ds(0, n, 2), :], scr[g][1, pl.ds(0, n, 2), :]],
                axis=1).astype(_bf)
            h_r = jnp.concatenate(
                [scr[g][0, pl.ds(1, n, 2), :], scr[g][1, pl.ds(1, n, 2), :]],
                axis=1).astype(_bf)
            new[g] = jnp.tanh(
                jnp.dot(h_l, bd3_ref[0], preferred_element_type=_f32)
                + jnp.dot(h_r, bd3_ref[1], preferred_element_type=_f32)
                + vs[g][n - 1:2 * n - 1])
        if d > 0:
            for g in range(G):
                scr[g][0, pl.ds(0, n), :] = new[g][:, :128]
                scr[g][1, pl.ds(0, n), :] = new[g][:, 128:]
    for g in range(G):
        o_ref[g] = new[g]


_S = np.zeros((G, JPP * N_FEAT, JF * N_FEAT), np.float32)
for _g in range(G):
    for _t in range(JF * N_FEAT):
        _S[_g, JF * N_FEAT * _g + _t, _t] = 1.0


def _bd4(x):
    a, b = x.shape[-2:]
    eye = jnp.eye(JF, dtype=x.dtype)
    t = (eye[..., :, None, :, None] * x[..., None, :, None, :])
    return t.reshape(*x.shape[:-2], JF * a, JF * b)


def kernel(content, Wu, bu, Wh, bh):
    c_w = (content.reshape(B, NODES, N_FEAT).transpose(1, 0, 2)
           .reshape(NODES, NPROG, 1, JPP * N_FEAT).astype(_bf))
    Wu_bd = _bd4(Wu.T)
    Wu_all = jnp.einsum('gkt,th->gkh', jnp.asarray(_S), Wu_bd).astype(_bf)
    BD3 = _bd4(Wh.T.reshape(3, N_HID, N_HID)).astype(_bf)
    bu_t = jnp.tile(bu, JF).reshape(1, W)
    bh_t = jnp.tile(bh, JF).reshape(1, W)

    out = pl.pallas_call(
        _body,
        grid=(NPROG,),
        in_specs=[
            pl.BlockSpec((NODES, 1, 1, JPP * N_FEAT), lambda i: (0, i, 0, 0)),
            pl.BlockSpec((G, JPP * N_FEAT, W), lambda i: (0, 0, 0)),
            pl.BlockSpec((3, W, W), lambda i: (0, 0, 0)),
            pl.BlockSpec((1, W), lambda i: (0, 0)),
            pl.BlockSpec((1, W), lambda i: (0, 0)),
        ],
        out_specs=pl.BlockSpec((G, 1, W), lambda i: (i, 0, 0)),
        out_shape=jax.ShapeDtypeStruct((NPROG * G, 1, W), jnp.float32),
        scratch_shapes=[pltpu.VMEM((2, LEAVES, 128), jnp.float32)
                        for _ in range(G)],
    )(c_w, Wu_all, BD3, bu_t, bh_t)
    return out.reshape(B, N_HID)


# speedup vs baseline: 1.1868x; 1.0106x over previous
import numpy as np
import jax
import jax.numpy as jnp
from jax.experimental import pallas as pl
from jax.experimental.pallas import tpu as pltpu

B = 128
DEPTH = 9
NODES = 2 ** (DEPTH + 1) - 1
LEAVES = 2 ** DEPTH
INNER = NODES - LEAVES
N_FEAT = 4
N_HID = 64
JF = 4
W = JF * N_HID
G = 8
JPP = JF * G
NPROG = B // JPP

_bf = jnp.bfloat16
_f32 = jnp.float32


def _body(c_ref, wu_ref, bd3_ref, bu_ref, bh_ref, o_ref, *scr):
    c = c_ref[:, 0, 0, :]
    vs = []
    for g in range(G):
        u = jnp.tanh(jnp.dot(c, wu_ref[g], preferred_element_type=_f32)
                     + bu_ref[...])
        v = (jnp.dot(u[:INNER].astype(_bf), bd3_ref[2],
                     preferred_element_type=_f32)
             + bh_ref[...])
        leaves = u[INNER:]
        scr[g][0, :, :] = leaves[:, :128]
        scr[g][1, :, :] = leaves[:, 128:]
        vs.append(v)
    new = [None] * G
    for d in range(DEPTH - 1, -1, -1):
        n = 2 ** d
        for g in range(G):
            h_l = jnp.concatenate(
                [scr[g][0, pl.ds(0, n, 2), :], scr[g][1, pl.ds(0, n, 2), :]],
                axis=1).astype(_bf)
            h_r = jnp.concatenate(
                [scr[g][0, pl.ds(1, n, 2), :], scr[g][1, pl.ds(1, n, 2), :]],
                axis=1).astype(_bf)
            new[g] = jnp.tanh(
                jnp.dot(h_l, bd3_ref[0], preferred_element_type=_f32)
                + jnp.dot(h_r, bd3_ref[1], preferred_element_type=_f32)
                + vs[g][n - 1:2 * n - 1])
        if d > 0:
            for g in range(G):
                scr[g][0, pl.ds(0, n), :] = new[g][:, :128]
                scr[g][1, pl.ds(0, n), :] = new[g][:, 128:]
    for g in range(G):
        o_ref[g] = new[g]


_S = np.zeros((G, JPP * N_FEAT, JF * N_FEAT), np.float32)
for _g in range(G):
    for _t in range(JF * N_FEAT):
        _S[_g, JF * N_FEAT * _g + _t, _t] = 1.0


def _bd4(x):
    a, b = x.shape[-2:]
    eye = jnp.eye(JF, dtype=x.dtype)
    t = (eye[..., :, None, :, None] * x[..., None, :, None, :])
    return t.reshape(*x.shape[:-2], JF * a, JF * b)


def kernel(content, Wu, bu, Wh, bh):
    c_w = (content.reshape(B, NODES, N_FEAT).transpose(1, 0, 2)
           .reshape(NODES, NPROG, 1, JPP * N_FEAT).astype(_bf))
    Wu_bd = _bd4(Wu.T)
    Wu_all = jnp.einsum('gkt,th->gkh', jnp.asarray(_S), Wu_bd).astype(_bf)
    BD3 = _bd4(Wh.T.reshape(3, N_HID, N_HID)).astype(_bf)
    bu_t = jnp.tile(bu, JF).reshape(1, W)
    bh_t = jnp.tile(bh, JF).reshape(1, W)

    out = pl.pallas_call(
        _body,
        grid=(NPROG,),
        in_specs=[
            pl.BlockSpec((NODES, 1, 1, JPP * N_FEAT), lambda i: (0, i, 0, 0)),
            pl.BlockSpec((G, JPP * N_FEAT, W), lambda i: (0, 0, 0)),
            pl.BlockSpec((3, W, W), lambda i: (0, 0, 0)),
            pl.BlockSpec((1, W), lambda i: (0, 0)),
            pl.BlockSpec((1, W), lambda i: (0, 0)),
        ],
        out_specs=pl.BlockSpec((G, 1, W), lambda i: (i, 0, 0)),
        out_shape=jax.ShapeDtypeStruct((NPROG * G, 1, W), jnp.float32),
        scratch_shapes=[pltpu.VMEM((2, LEAVES, 128), jnp.float32)
                        for _ in range(G)],
    )(c_w, Wu_all, BD3, bu_t, bh_t)
    return out.reshape(B, N_HID)
